# Initial kernel scaffold; baseline (speedup 1.0000x reference)
import jax
import jax.numpy as jnp
from jax.experimental import pallas as pl


def kernel(x, batch, table_0, table_1, table_2, table_3, table_4, table_5, table_6, table_7, table_8, W, b):
    raise NotImplementedError("write your pallas kernel here")


# speedup vs baseline: 65.7181x; 65.7181x over previous
import functools

import jax
import jax.numpy as jnp
from jax import lax
from jax.experimental import pallas as pl
from jax.experimental.pallas import tpu as pltpu
from jax.experimental.pallas import tpu_sc as plsc

_G = 512
_N = 100000
_NW = 32
_NPAD = 100352
_CHUNK = _NPAD // _NW
_NVEC = _CHUNK // 16
_NSEG = 544

_D012 = (119, 4, 12)
_D345 = (12, 10, 6)
_D678 = (6, 2, 2)


def _project_body(t0, t1, t2, t3, t4, t5, t6, t7, t8, w, v012, v345, v678):
    p = [jnp.dot(t[...], w[...])[:, 0]
         for t in (t0, t1, t2, t3, t4, t5, t6, t7, t8)]
    v012[...] = p[0][:, None, None] + p[1][None, :, None] + p[2][None, None, :]
    v345[...] = p[3][:, None, None] + p[4][None, :, None] + p[5][None, None, :]
    v678[...] = p[6][:, None, None] + p[7][None, :, None] + p[8][None, None, :]


def _pool_body(xT, batch, v012, v345, v678, out,
               xv, bv, t012, t345, t678, acc_s, acc_c, sem):
    wid = lax.axis_index("c") * 16 + lax.axis_index("s")
    base = wid * _CHUNK

    copies = []
    for f in range(9):
        copies.append(pltpu.async_copy(
            xT.at[pl.ds(f * _NPAD + base, _CHUNK)],
            xv.at[pl.ds(f * _CHUNK, _CHUNK)], sem))
    copies.append(pltpu.async_copy(batch.at[pl.ds(base, _CHUNK)], bv, sem))
    copies.append(pltpu.async_copy(v012, t012, sem))
    copies.append(pltpu.async_copy(v345, t345, sem))
    copies.append(pltpu.async_copy(v678, t678, sem))

    zeros = jnp.zeros((16,), jnp.float32)

    def zero_body(k, _):
        acc_s[pl.ds(k * 16, 16)] = zeros
        acc_c[pl.ds(k * 16, 16)] = zeros
        return 0

    lax.fori_loop(0, _NSEG // 16, zero_body, 0)

    for c in copies:
        c.wait()

    ones = jnp.full((16,), 1.0, jnp.float32)

    def body(j, _):
        off = j * 16
        idx = [xv[pl.ds(f * _CHUNK + off, 16)] for f in range(9)]
        i012 = (idx[0] * 4 + idx[1]) * 12 + idx[2]
        i345 = (idx[3] * 10 + idx[4]) * 6 + idx[5]
        i678 = (idx[6] * 2 + idx[7]) * 2 + idx[8]
        s = plsc.load_gather(t012, [i012])
        s = s + plsc.load_gather(t345, [i345])
        s = s + plsc.load_gather(t678, [i678])
        g = bv[pl.ds(off, 16)]
        plsc.addupdate_scatter(acc_s, [g], s)
        plsc.addupdate_scatter(acc_c, [g], ones)
        return 0

    lax.fori_loop(0, _NVEC, body, 0)

    pltpu.sync_copy(acc_s, out.at[wid, 0])
    pltpu.sync_copy(acc_c, out.at[wid, 1])


def _finish_body(p_ref, b_ref, o_ref):
    tot = jnp.sum(p_ref[...], axis=0)
    sums = tot[0, :_G]
    counts = tot[1, :_G]
    o_ref[...] = (sums / jnp.maximum(counts, 1.0) + b_ref[0, 0])[None, :]


def kernel(x, batch, table_0, table_1, table_2, table_3, table_4,
           table_5, table_6, table_7, table_8, W, b):
    pad = _NPAD - x.shape[0]
    xT = jnp.pad(x, ((0, pad), (0, 0))).T.reshape(-1)
    batch_p = jnp.concatenate(
        [batch, jnp.full((pad,), _G, jnp.int32)])

    v012, v345, v678 = pl.pallas_call(
        _project_body,
        out_shape=[jax.ShapeDtypeStruct(_D012, jnp.float32),
                   jax.ShapeDtypeStruct(_D345, jnp.float32),
                   jax.ShapeDtypeStruct(_D678, jnp.float32)],
    )(table_0, table_1, table_2, table_3, table_4, table_5, table_6,
      table_7, table_8, W)

    pool = pl.kernel(
        _pool_body,
        out_type=jax.ShapeDtypeStruct((_NW, 2, _NSEG), jnp.float32),
        mesh=plsc.VectorSubcoreMesh(core_axis_name="c", subcore_axis_name="s"),
        compiler_params=pltpu.CompilerParams(needs_layout_passes=False),
        scratch_types=[
            pltpu.VMEM((9 * _CHUNK,), jnp.int32),
            pltpu.VMEM((_CHUNK,), jnp.int32),
            pltpu.VMEM((119 * 4 * 12,), jnp.float32),
            pltpu.VMEM((12 * 10 * 6,), jnp.float32),
            pltpu.VMEM((6 * 2 * 2,), jnp.float32),
            pltpu.VMEM((_NSEG,), jnp.float32),
            pltpu.VMEM((_NSEG,), jnp.float32),
            pltpu.SemaphoreType.DMA,
        ],
    )
    partials = pool(xT, batch_p, v012.reshape(-1), v345.reshape(-1),
                    v678.reshape(-1))

    out = pl.pallas_call(
        _finish_body,
        out_shape=jax.ShapeDtypeStruct((1, _G), jnp.float32),
    )(partials, b.reshape(1, 1))
    return out.reshape(_G, 1)


# speedup vs baseline: 73.3449x; 1.1161x over previous
import functools

import jax
import jax.numpy as jnp
from jax import lax
from jax.experimental import pallas as pl
from jax.experimental.pallas import tpu as pltpu
from jax.experimental.pallas import tpu_sc as plsc

_G = 512
_N = 100000
_NW = 32
_NPAD = 100352
_CHUNK = _NPAD // _NW
_NVEC = _CHUNK // 16
_NSEG = 544
_PITCH = 545

_D012 = (119, 4, 12)
_D345 = (12, 10, 6)
_D678 = (6, 2, 2)


def _project_body(t0, t1, t2, t3, t4, t5, t6, t7, t8, w, v012, v345, v678):
    p = [jnp.dot(t[...], w[...])[:, 0]
         for t in (t0, t1, t2, t3, t4, t5, t6, t7, t8)]
    v012[...] = p[0][:, None, None] + p[1][None, :, None] + p[2][None, None, :]
    v345[...] = p[3][:, None, None] + p[4][None, :, None] + p[5][None, None, :]
    v678[...] = p[6][:, None, None] + p[7][None, :, None] + p[8][None, None, :]


def _pool_body(xT, batch, v012, v345, v678, out,
               xv, bv, t012, t345, t678, acc_s, acc_c, obuf, sem):
    wid = lax.axis_index("c") * 16 + lax.axis_index("s")
    base = wid * _CHUNK

    copies = []
    for f in range(9):
        copies.append(pltpu.async_copy(
            xT.at[pl.ds(f * _NPAD + base, _CHUNK)],
            xv.at[pl.ds(f * _CHUNK, _CHUNK)], sem))
    copies.append(pltpu.async_copy(batch.at[pl.ds(base, _CHUNK)], bv, sem))
    copies.append(pltpu.async_copy(v012, t012, sem))
    copies.append(pltpu.async_copy(v345, t345, sem))
    copies.append(pltpu.async_copy(v678, t678, sem))

    zeros = jnp.zeros((16,), jnp.float32)

    def zero_body(k, _):
        acc_s[pl.ds(k * 16, 16)] = zeros
        acc_c[pl.ds(k * 16, 16)] = zeros
        return 0

    lax.fori_loop(0, _PITCH, zero_body, 0)

    for c in copies:
        c.wait()

    ones = jnp.full((16,), 1.0, jnp.float32)
    lane_base = lax.iota(jnp.int32, 16) * _PITCH

    def body(j, _):
        off = j * 16
        idx = [xv[pl.ds(f * _CHUNK + off, 16)] for f in range(9)]
        i012 = (idx[0] * 4 + idx[1]) * 12 + idx[2]
        i345 = (idx[3] * 10 + idx[4]) * 6 + idx[5]
        i678 = (idx[6] * 2 + idx[7]) * 2 + idx[8]
        s = plsc.load_gather(t012, [i012])
        s = s + plsc.load_gather(t345, [i345])
        s = s + plsc.load_gather(t678, [i678])
        g = bv[pl.ds(off, 16)] + lane_base
        plsc.addupdate_scatter(acc_s, [g], s)
        plsc.addupdate_scatter(acc_c, [g], ones)
        return 0

    lax.fori_loop(0, _NVEC, body, 0)

    def red_body(k, _):
        off = k * 16
        ssum = acc_s[pl.ds(off, 16)]
        csum = acc_c[pl.ds(off, 16)]
        for l in range(1, 16):
            ssum = ssum + acc_s[pl.ds(l * _PITCH + off, 16)]
            csum = csum + acc_c[pl.ds(l * _PITCH + off, 16)]
        obuf[pl.ds(off, 16)] = ssum
        obuf[pl.ds(_NSEG + off, 16)] = csum
        return 0

    lax.fori_loop(0, _NSEG // 16, red_body, 0)

    pltpu.sync_copy(obuf, out.at[wid])


def _finish_body(p_ref, b_ref, o_ref):
    tot = jnp.sum(p_ref[...], axis=0)
    sums = tot[:_G]
    counts = tot[_NSEG:_NSEG + _G]
    o_ref[...] = (sums / jnp.maximum(counts, 1.0) + b_ref[0, 0])[None, :]


def kernel(x, batch, table_0, table_1, table_2, table_3, table_4,
           table_5, table_6, table_7, table_8, W, b):
    pad = _NPAD - x.shape[0]
    xT = jnp.pad(x, ((0, pad), (0, 0))).T.reshape(-1)
    batch_p = jnp.concatenate(
        [batch, jnp.full((pad,), _G, jnp.int32)])

    v012, v345, v678 = pl.pallas_call(
        _project_body,
        out_shape=[jax.ShapeDtypeStruct(_D012, jnp.float32),
                   jax.ShapeDtypeStruct(_D345, jnp.float32),
                   jax.ShapeDtypeStruct(_D678, jnp.float32)],
    )(table_0, table_1, table_2, table_3, table_4, table_5, table_6,
      table_7, table_8, W)

    pool = pl.kernel(
        _pool_body,
        out_type=jax.ShapeDtypeStruct((_NW, 2 * _NSEG), jnp.float32),
        mesh=plsc.VectorSubcoreMesh(core_axis_name="c", subcore_axis_name="s"),
        compiler_params=pltpu.CompilerParams(needs_layout_passes=False),
        scratch_types=[
            pltpu.VMEM((9 * _CHUNK,), jnp.int32),
            pltpu.VMEM((_CHUNK,), jnp.int32),
            pltpu.VMEM((119 * 4 * 12,), jnp.float32),
            pltpu.VMEM((12 * 10 * 6,), jnp.float32),
            pltpu.VMEM((6 * 2 * 2,), jnp.float32),
            pltpu.VMEM((16 * _PITCH,), jnp.float32),
            pltpu.VMEM((16 * _PITCH,), jnp.float32),
            pltpu.VMEM((2 * _NSEG,), jnp.float32),
            pltpu.SemaphoreType.DMA,
        ],
    )
    partials = pool(xT, batch_p, v012.reshape(-1), v345.reshape(-1),
                    v678.reshape(-1))

    out = pl.pallas_call(
        _finish_body,
        out_shape=jax.ShapeDtypeStruct((1, _G), jnp.float32),
    )(partials, b.reshape(1, 1))
    return out.reshape(_G, 1)
